# baseline (speedup 1.0000x reference)
import dataclasses
import functools

import jax
import jax.numpy as jnp
from jax import lax
from jax.experimental import pallas as pl
from jax.experimental.pallas import tpu as pltpu
from jax.experimental.pallas import tpu_sc as plsc

N_EMB = 1000000
EMB_DIM = 64
CLS = 2
CLS_PAD = 16
BATCH = 16384
HIST = 200

NC = 2
NS = 16
L = 16
NW = NC * NS
B_PER_W = BATCH // NW
G = 16
GROUPS = B_PER_W // G
IDX_PER_G = G * HIST
IDX_W = 128
IDX_ROWS = IDX_PER_G // IDX_W
UNROLL = 8

MM_BLK = 32768
MM_SUB = MM_BLK // 8
MM_SUB_SHIFT = 12
MM_GRID = -(-N_EMB // MM_BLK)
TW_ROWS = MM_GRID * MM_BLK


def _mm_body(t_ref, w_ref, o_ref):
    t16 = t_ref[...].astype(jnp.bfloat16)
    acc = None
    for j in range(8):
        part = lax.dot_general(
            t16[:, j * MM_SUB:(j + 1) * MM_SUB],
            w_ref[j * EMB_DIM:(j + 1) * EMB_DIM, :],
            dimension_numbers=(((0,), (0,)), ((), ())),
            preferred_element_type=jnp.float32,
        )
        acc = part if acc is None else acc + part
    o_ref[...] = acc


def _fold_table(tableT, Wexp):
    return pl.pallas_call(
        _mm_body,
        grid=(MM_GRID,),
        in_specs=[
            pl.BlockSpec((EMB_DIM, MM_BLK), lambda i: (0, i)),
            pl.BlockSpec((8 * EMB_DIM, 128), lambda i: (0, 0)),
        ],
        out_specs=pl.BlockSpec((MM_SUB, 128), lambda i: (i, 0)),
        out_shape=jax.ShapeDtypeStruct((MM_GRID * MM_SUB, 128), jnp.float32),
        compiler_params=pltpu.CompilerParams(fuse_transposed_lhs_in_matmul=True),
    )(tableT, Wexp)


def _sc_gather_sum(xT, tableW, b2):
    mesh = plsc.VectorSubcoreMesh(core_axis_name="c", subcore_axis_name="s")
    cp = pltpu.CompilerParams(
        needs_layout_passes=False,
        use_tc_tiling_on_sc=False,
    )

    @functools.partial(
        pl.kernel,
        out_type=jax.ShapeDtypeStruct((BATCH, CLS), jnp.float32),
        mesh=mesh,
        compiler_params=cp,
        scratch_types=[
            pltpu.VMEM((HIST, G), jnp.int32),
            pltpu.VMEM((4, IDX_ROWS, IDX_W), jnp.int32),
            pltpu.VMEM((2, IDX_PER_G, CLS_PAD), jnp.float32),
            pltpu.VMEM((B_PER_W, CLS), jnp.float32),
            pltpu.VMEM((CLS, L), jnp.float32),
            pltpu.SemaphoreType.DMA,
            pltpu.SemaphoreType.DMA,
        ],
    )
    def k(x_hbm, tw_hbm, b_hbm, out_hbm, raw_v, idx_v, rows_v, out_v, b_v,
          sem0, sem1):
        wid = lax.axis_index("s") * NC + lax.axis_index("c")
        col_base = wid * B_PER_W
        lanes = lax.iota(jnp.int32, L)
        col0 = jnp.zeros((L,), jnp.int32)
        col1 = jnp.ones((L,), jnp.int32)
        sems = (sem0, sem1)
        pltpu.sync_copy(b_hbm, b_v)

        def prep_idx(g, ibuf):
            pltpu.sync_copy(x_hbm.at[:, pl.ds(col_base + g * G, G)], raw_v)

            @pl.loop(0, IDX_ROWS)
            def _(rr):
                for kk in range(IDX_W // L):
                    v = raw_v[rr * (IDX_W // L) + kk, :]
                    idx_v[ibuf, rr, pl.ds(kk * L, L)] = (
                        (v & -MM_BLK)
                        | ((v & (MM_SUB - 1)) << 3)
                        | ((v >> MM_SUB_SHIFT) & 7))

        def fire(ibuf, buf, sem):
            for j in range(IDX_ROWS):
                pltpu.async_copy(
                    tw_hbm.at[idx_v.at[ibuf, j]],
                    rows_v.at[buf, pl.ds(j * IDX_W, IDX_W)],
                    sem,
                )

        def drain(buf, sem):
            pltpu.make_async_copy(
                tw_hbm.at[pl.ds(0, IDX_PER_G)], rows_v.at[buf], sem).wait()

        def reduce(g, buf):
            def body(i, accs):
                a0, a1 = accs
                for kk in range(UNROLL):
                    r = (i * UNROLL + kk) * L + lanes
                    a0 = a0 + plsc.load_gather(rows_v.at[buf], [r, col0])
                    a1 = a1 + plsc.load_gather(rows_v.at[buf], [r, col1])
                return (a0, a1)

            a0, a1 = lax.fori_loop(0, HIST // UNROLL, body,
                                   (b_v[0, :], b_v[1, :]))
            row_idx = g * G + lanes
            plsc.store_scatter(out_v, [row_idx, col0], a0)
            plsc.store_scatter(out_v, [row_idx, col1], a1)

        prep_idx(0, 0)
        prep_idx(1, 1)
        fire(0, 0, sems[0])

        @pl.loop(0, GROUPS, step=4)
        def _(g):
            for p in range(4):
                gg = g + p
                nxt = gg + 1

                @pl.when(nxt < GROUPS)
                def _():
                    fire((p + 1) % 4, (p + 1) % 2, sems[(p + 1) % 2])

                drain(p % 2, sems[p % 2])
                reduce(gg, p % 2)

                pre = gg + 2

                @pl.when(pre < GROUPS)
                def _():
                    prep_idx(pre, (p + 2) % 4)

        pltpu.sync_copy(out_v, out_hbm.at[pl.ds(wid * B_PER_W, B_PER_W)])

    return k(xT, tableW, b2)


def kernel(x, table, W, b):
    xT = x.T.astype(jnp.int32)
    Wexp = jnp.zeros((8 * EMB_DIM, 128), jnp.bfloat16)
    for j in range(8):
        Wexp = Wexp.at[j * EMB_DIM:(j + 1) * EMB_DIM,
                       j * CLS_PAD:j * CLS_PAD + CLS].set(
                           (W.T * (1.0 / HIST)).astype(jnp.bfloat16))
    b2 = jnp.broadcast_to(b.reshape(CLS, 1), (CLS, L)).astype(jnp.float32)
    tableW = _fold_table(table.T, Wexp).reshape(TW_ROWS, CLS_PAD)
    return _sc_gather_sum(xT, tableW, b2)


# speedup vs baseline: 1.0070x; 1.0070x over previous
import functools

import jax
import jax.numpy as jnp
from jax import lax
from jax.experimental import pallas as pl
from jax.experimental.pallas import tpu as pltpu
from jax.experimental.pallas import tpu_sc as plsc

N_EMB = 1000000
EMB_DIM = 64
CLS = 2
CLS_PAD = 16
BATCH = 16384
HIST = 200

NC = 2
NS = 16
L = 16
NW = NC * NS
B_PER_W = BATCH // NW
G = 16
GROUPS = B_PER_W // G
IDX_PER_G = G * HIST
IDX_W = 128
IDX_ROWS = IDX_PER_G // IDX_W
UNROLL = 8

MM_BLK = 32768
MM_SUB = MM_BLK // 8
MM_SUB_SHIFT = 12
MM_GRID = -(-N_EMB // MM_BLK)
TW_ROWS = MM_GRID * MM_BLK


def _mm_body(t_ref, w_ref, o_ref):
    t16 = t_ref[...].astype(jnp.bfloat16)
    acc = None
    for j in range(8):
        part = lax.dot_general(
            t16[:, j * MM_SUB:(j + 1) * MM_SUB],
            w_ref[j * EMB_DIM:(j + 1) * EMB_DIM, :],
            dimension_numbers=(((0,), (0,)), ((), ())),
            preferred_element_type=jnp.float32,
        )
        acc = part if acc is None else acc + part
    o_ref[...] = acc


def _fold_table(tableT, Wexp):
    return pl.pallas_call(
        _mm_body,
        grid=(MM_GRID,),
        in_specs=[
            pl.BlockSpec((EMB_DIM, MM_BLK), lambda i: (0, i)),
            pl.BlockSpec((8 * EMB_DIM, 128), lambda i: (0, 0)),
        ],
        out_specs=pl.BlockSpec((MM_SUB, 128), lambda i: (i, 0)),
        out_shape=jax.ShapeDtypeStruct((MM_GRID * MM_SUB, 128), jnp.float32),
        compiler_params=pltpu.CompilerParams(fuse_transposed_lhs_in_matmul=True),
    )(tableT, Wexp)


def _sc_gather_sum(xT, tableW, b2):
    mesh = plsc.VectorSubcoreMesh(core_axis_name="c", subcore_axis_name="s")
    cp = pltpu.CompilerParams(
        needs_layout_passes=False,
        use_tc_tiling_on_sc=False,
    )

    @functools.partial(
        pl.kernel,
        out_type=jax.ShapeDtypeStruct((BATCH, CLS), jnp.float32),
        mesh=mesh,
        compiler_params=cp,
        scratch_types=[
            pltpu.VMEM((HIST, G), jnp.int32),
            pltpu.VMEM((2, IDX_ROWS, IDX_W), jnp.int32),
            pltpu.VMEM((2, IDX_PER_G, CLS_PAD), jnp.float32),
            pltpu.VMEM((B_PER_W, CLS), jnp.float32),
            pltpu.VMEM((CLS, L), jnp.float32),
            pltpu.SemaphoreType.DMA,
            pltpu.SemaphoreType.DMA,
        ],
    )
    def k(x_hbm, tw_hbm, b_hbm, out_hbm, raw_v, idx_v, rows_v, out_v, b_v,
          sem0, sem1):
        wid = lax.axis_index("s") * NC + lax.axis_index("c")
        col_base = wid * B_PER_W
        lanes = lax.iota(jnp.int32, L)
        col0 = jnp.zeros((L,), jnp.int32)
        col1 = jnp.ones((L,), jnp.int32)
        sems = (sem0, sem1)
        pltpu.sync_copy(b_hbm, b_v)

        def prep(g, buf, sem):
            pltpu.sync_copy(x_hbm.at[:, pl.ds(col_base + g * G, G)], raw_v)

            @pl.loop(0, IDX_ROWS)
            def _(rr):
                for kk in range(IDX_W // L):
                    v = raw_v[rr * (IDX_W // L) + kk, :]
                    idx_v[buf, rr, pl.ds(kk * L, L)] = (
                        (v & -MM_BLK)
                        | ((v & (MM_SUB - 1)) << 3)
                        | ((v >> MM_SUB_SHIFT) & 7))

            for j in range(IDX_ROWS):
                pltpu.async_copy(
                    tw_hbm.at[idx_v.at[buf, j]],
                    rows_v.at[buf, pl.ds(j * IDX_W, IDX_W)],
                    sem,
                )

        def drain(buf, sem):
            pltpu.make_async_copy(
                tw_hbm.at[pl.ds(0, IDX_PER_G)], rows_v.at[buf], sem).wait()

        def reduce(g, buf):
            def body(i, accs):
                a0, a1 = accs
                for kk in range(UNROLL):
                    r = (i * UNROLL + kk) * L + lanes
                    a0 = a0 + plsc.load_gather(rows_v.at[buf], [r, col0])
                    a1 = a1 + plsc.load_gather(rows_v.at[buf], [r, col1])
                return (a0, a1)

            a0, a1 = lax.fori_loop(0, HIST // UNROLL, body,
                                   (b_v[0, :], b_v[1, :]))
            row_idx = g * G + lanes
            plsc.store_scatter(out_v, [row_idx, col0], a0)
            plsc.store_scatter(out_v, [row_idx, col1], a1)

        prep(0, 0, sems[0])

        @pl.loop(0, GROUPS, step=2)
        def _(g):
            for p in range(2):
                nxt = g + p + 1

                @pl.when(nxt < GROUPS)
                def _():
                    prep(nxt, (p + 1) % 2, sems[(p + 1) % 2])

                drain(p, sems[p])
                reduce(g + p, p)

        pltpu.sync_copy(out_v, out_hbm.at[pl.ds(wid * B_PER_W, B_PER_W)])

    return k(xT, tableW, b2)


def kernel(x, table, W, b):
    xT = x.T.astype(jnp.int32)
    Wexp = jnp.zeros((8 * EMB_DIM, 128), jnp.bfloat16)
    for j in range(8):
        Wexp = Wexp.at[j * EMB_DIM:(j + 1) * EMB_DIM,
                       j * CLS_PAD:j * CLS_PAD + CLS].set(
                           (W.T * (1.0 / HIST)).astype(jnp.bfloat16))
    b2 = jnp.broadcast_to(b.reshape(CLS, 1), (CLS, L)).astype(jnp.float32)
    tableW = _fold_table(table.T, Wexp).reshape(TW_ROWS, CLS_PAD)
    return _sc_gather_sum(xT, tableW, b2)
